# Initial kernel scaffold; baseline (speedup 1.0000x reference)
import jax
import jax.numpy as jnp
from jax.experimental import pallas as pl


def kernel(x, edge_index, W1, b1, W2, b2):
    raise NotImplementedError("write your pallas kernel here")


# speedup vs baseline: 12.7176x; 12.7176x over previous
import functools

import jax
import jax.numpy as jnp
from jax import lax
from jax.experimental import pallas as pl
from jax.experimental.pallas import tpu as pltpu
from jax.experimental.pallas import tpu_sc as plsc

N = 10000
D = 128
E = 320000

NC = 2
NS = 16
NW = NC * NS
EPT = E // NW
CH = 80
NCH = EPT // CH

_SC_MESH = plsc.VectorSubcoreMesh(
    core_axis_name="c", subcore_axis_name="s", num_cores=NC, num_subcores=NS)


@functools.partial(
    pl.kernel,
    out_type=jax.ShapeDtypeStruct((NC * N,), jnp.float32),
    mesh=_SC_MESH,
    scratch_types=[
        pltpu.VMEM((CH,), jnp.int32),
        pltpu.VMEM((CH,), jnp.float32),
        pltpu.VMEM((1000,), jnp.float32),
        pltpu.VMEM_SHARED((N,), jnp.float32),
    ],
)
def _sc_degree(dst_hbm, ones_hbm, zeros_hbm, out_hbm, dst_v, ones_v,
               stage_v, acc_sh):
    c = lax.axis_index("c")
    s = lax.axis_index("s")

    @pl.when(s < 10)
    def _zero():
        pltpu.sync_copy(zeros_hbm, stage_v)
        pltpu.sync_copy(stage_v, acc_sh.at[pl.ds(s * 1000, 1000)])

    pltpu.sync_copy(ones_hbm, ones_v)
    plsc.subcore_barrier()

    ebase = (c * NS + s) * EPT

    def body(i, carry):
        off = ebase + i * CH
        pltpu.sync_copy(dst_hbm.at[pl.ds(off, CH)], dst_v)
        pltpu.sync_copy(ones_v, acc_sh.at[dst_v], add=True)
        return carry

    lax.fori_loop(0, NCH, body, 0, unroll=False)
    plsc.subcore_barrier()

    @pl.when(s < 10)
    def _writeout():
        pltpu.sync_copy(acc_sh.at[pl.ds(s * 1000, 1000)], stage_v)
        pltpu.sync_copy(stage_v, out_hbm.at[pl.ds(c * N + s * 1000, 1000)])


@functools.partial(
    pl.kernel,
    out_type=jax.ShapeDtypeStruct((NC, N, D), jnp.float32),
    mesh=_SC_MESH,
    scratch_types=[
        pltpu.VMEM((CH,), jnp.int32),
        pltpu.VMEM((CH,), jnp.int32),
        pltpu.VMEM((CH, D), jnp.float32),
        pltpu.VMEM((200, D), jnp.float32),
        pltpu.VMEM_SHARED((N, D), jnp.float32),
        pltpu.SemaphoreType.DMA,
    ],
)
def _sc_aggregate(g_hbm, src_hbm, dst_hbm, zeros_hbm, out_hbm,
                  src_v, dst_v, rows_v, stage_v, acc_sh, sem):
    c = lax.axis_index("c")
    s = lax.axis_index("s")

    @pl.when(s < 10)
    def _zero():
        pltpu.sync_copy(zeros_hbm, stage_v)
        for j in range(5):
            pltpu.sync_copy(stage_v,
                            acc_sh.at[pl.ds(s * 1000 + j * 200, 200)])

    plsc.subcore_barrier()

    ebase = (c * NS + s) * EPT

    def body(i, carry):
        off = ebase + i * CH
        pltpu.sync_copy(src_hbm.at[pl.ds(off, CH)], src_v)
        pltpu.sync_copy(dst_hbm.at[pl.ds(off, CH)], dst_v)
        pltpu.async_copy(g_hbm.at[src_v], rows_v, sem).wait()
        pltpu.sync_copy(rows_v, acc_sh.at[dst_v], add=True)
        return carry

    lax.fori_loop(0, NCH, body, 0, unroll=False)
    plsc.subcore_barrier()

    @pl.when(s < 10)
    def _writeout():
        for j in range(5):
            row = s * 1000 + j * 200
            pltpu.sync_copy(acc_sh.at[pl.ds(row, 200)], stage_v)
            pltpu.sync_copy(stage_v, out_hbm.at[c, pl.ds(row, 200)])


_BM = 2000


def _tc_scale_matmul_body(degA, degB, x_ref, w_ref, g_ref):
    dinv = lax.rsqrt(degA[...] + degB[...] + 1.0)
    h = jnp.dot(x_ref[...], w_ref[...], preferred_element_type=jnp.float32)
    g_ref[...] = h * dinv


def _tc_mid_body(degA, degB, aggA, aggB, g_ref, b_ref, w_ref, out_ref):
    dinv = lax.rsqrt(degA[...] + degB[...] + 1.0)
    h = (aggA[...] + aggB[...] + g_ref[...]) * dinv + b_ref[...]
    h = jnp.maximum(h, 0.0)
    out_ref[...] = jnp.dot(
        h, w_ref[...], preferred_element_type=jnp.float32) * dinv


def _tc_final_body(degA, degB, aggA, aggB, g_ref, b_ref, out_ref):
    dinv = lax.rsqrt(degA[...] + degB[...] + 1.0)
    h = (aggA[...] + aggB[...] + g_ref[...]) * dinv + b_ref[...]
    out_ref[...] = jnp.maximum(h, 0.0)


_col_spec = pl.BlockSpec((_BM, 1), lambda i: (i, 0))
_row_spec = pl.BlockSpec((_BM, D), lambda i: (i, 0))
_w_spec = pl.BlockSpec((D, D), lambda i: (0, 0))
_b_spec = pl.BlockSpec((1, D), lambda i: (0, 0))
_GRID = (N // _BM,)
_out_nd = jax.ShapeDtypeStruct((N, D), jnp.float32)

_tc_scale_matmul = pl.pallas_call(
    _tc_scale_matmul_body, grid=_GRID,
    in_specs=[_col_spec, _col_spec, _row_spec, _w_spec],
    out_specs=_row_spec, out_shape=_out_nd)

_tc_mid = pl.pallas_call(
    _tc_mid_body, grid=_GRID,
    in_specs=[_col_spec, _col_spec, _row_spec, _row_spec, _row_spec,
              _b_spec, _w_spec],
    out_specs=_row_spec, out_shape=_out_nd)

_tc_final = pl.pallas_call(
    _tc_final_body, grid=_GRID,
    in_specs=[_col_spec, _col_spec, _row_spec, _row_spec, _row_spec, _b_spec],
    out_specs=_row_spec, out_shape=_out_nd)


def kernel(x, edge_index, W1, b1, W2, b2):
    src = edge_index[0]
    dst = edge_index[1]
    zeros_n = jnp.zeros((1000,), jnp.float32)
    zeros_nd = jnp.zeros((200, D), jnp.float32)
    ones_ch = jnp.ones((CH,), jnp.float32)
    b1r = b1.reshape(1, D)
    b2r = b2.reshape(1, D)

    degp = _sc_degree(dst, ones_ch, zeros_n).reshape(NC, N)
    degA = degp[0][:, None]
    degB = degp[1][:, None]

    g1 = _tc_scale_matmul(degA, degB, x, W1)
    agg1 = _sc_aggregate(g1, src, dst, zeros_nd)
    g2 = _tc_mid(degA, degB, agg1[0], agg1[1], g1, b1r, W2)
    agg2 = _sc_aggregate(g2, src, dst, zeros_nd)
    out = _tc_final(degA, degB, agg2[0], agg2[1], g2, b2r)
    return out
